# baseline (speedup 1.0000x reference)
import functools
import math

import jax
import jax.numpy as jnp
from jax import lax
from jax.experimental import pallas as pl
from jax.experimental.pallas import tpu as pltpu
from jax.experimental.pallas import tpu_sc as plsc

_NB = 32
_NH = 16
_S = 2048
_W = 4224
_THR = [math.ceil(16 * 8 ** (t / 16)) for t in range(1, 16)]

_HALF = _W // 2
_CHUNKS = _HALF // 16


def _sc_body(table_hbm, delta_hbm, out_hbm, table_v, delta_v, vals_v):
    nc = plsc.get_sparse_core_info().num_cores
    wid = lax.axis_index("s") * nc + lax.axis_index("c")
    hh = wid // 2
    part = wid % 2
    pltpu.sync_copy(table_hbm, table_v)
    pltpu.sync_copy(delta_hbm, delta_v)
    vdelta = delta_v[...]
    hvec = jnp.full((16,), 0, jnp.int32) + hh
    base = part * _HALF
    lane = lax.iota(jnp.int32, 16)

    def chunk(k, acc):
        c = lane + (base + k * 16)
        n = jnp.abs(c + (vdelta - _S))
        large = jnp.full((16,), 16, jnp.int32)
        for t in _THR:
            large += (n >= t).astype(jnp.int32)
        bucket = jnp.where(n < 16, n, large)
        vals_v[pl.ds(k * 16, 16)] = plsc.load_gather(table_v, [bucket * _NH + hvec])
        return acc

    lax.fori_loop(0, _CHUNKS, chunk, 0)
    pltpu.sync_copy(vals_v, out_hbm.at[pl.ds(hh * _W + base, _HALF)])


_sc_vals = functools.partial(
    pl.kernel,
    out_type=jax.ShapeDtypeStruct((_NH * _W,), jnp.float32),
    mesh=plsc.VectorSubcoreMesh(core_axis_name="c", subcore_axis_name="s"),
    scratch_types=[
        pltpu.VMEM((_NB * _NH,), jnp.float32),
        pltpu.VMEM((16,), jnp.int32),
        pltpu.VMEM((_HALF,), jnp.float32),
    ],
    compiler_params=pltpu.CompilerParams(needs_layout_passes=False),
)(_sc_body)


def _copies(rolled_ref, out_hbm, sem, bi, h):
    for rb in range(_S // 128):
        yield pltpu.make_async_copy(
            rolled_ref.at[bi, :, pl.ds(_S - 128 * rb, _S)],
            out_hbm.at[0, h, pl.ds(128 * rb, 128), :],
            sem,
        )


def _tc_body(vals_ref, out_hbm, rolled_ref, sem0, sem1):
    h = pl.program_id(0)
    even = (h % 2) == 0

    def run(bi, sem):
        @pl.when(h >= 2)
        def _drain_prev():
            for cp in _copies(rolled_ref, out_hbm, sem, bi, h - 2):
                cp.wait()

        x = jnp.broadcast_to(vals_ref[0], (128, _W))
        m = jax.lax.broadcasted_iota(jnp.int32, (128, _W), 0)
        for t in range(7):
            sh = 1 << t
            x = jnp.where((m & sh) != 0, pltpu.roll(x, sh, 1), x)
        rolled_ref[bi] = x
        for cp in _copies(rolled_ref, out_hbm, sem, bi, h):
            cp.start()

    @pl.when(even)
    def _even():
        run(0, sem0)

    @pl.when(jnp.logical_not(even))
    def _odd():
        run(1, sem1)

    @pl.when(h == _NH - 1)
    def _final_drain():
        for cp in _copies(rolled_ref, out_hbm, sem0, 0, h - 1):
            cp.wait()
        for cp in _copies(rolled_ref, out_hbm, sem1, 1, h):
            cp.wait()


def kernel(table, seq_len, past_key_values_length):
    delta = (jnp.asarray(seq_len).astype(jnp.int32) - _S
             - jnp.asarray(past_key_values_length).astype(jnp.int32))
    vals = _sc_vals(table.astype(jnp.float32).reshape(_NB * _NH),
                    jnp.full((16,), delta, jnp.int32))
    return pl.pallas_call(
        _tc_body,
        grid=(_NH,),
        in_specs=[
            pl.BlockSpec((1, 1, _W), lambda h: (h, 0, 0)),
        ],
        out_specs=pl.BlockSpec(memory_space=pl.ANY),
        out_shape=jax.ShapeDtypeStruct((1, _NH, _S, _S), jnp.float32),
        scratch_shapes=[
            pltpu.VMEM((2, 128, _W), jnp.float32),
            pltpu.SemaphoreType.DMA,
            pltpu.SemaphoreType.DMA,
        ],
        compiler_params=pltpu.CompilerParams(
            dimension_semantics=("arbitrary",),
        ),
    )(vals.reshape(_NH, 1, _W))
